# baseline (speedup 1.0000x reference)
import jax
import jax.numpy as jnp
from jax.experimental import pallas as pl
from jax.experimental.pallas import tpu as pltpu

_N = 100000
_D = 128
_CHUNK = 2000
_NBUF = 3
_NCHUNKS = _N // _CHUNK
_ROUNDS = _NCHUNKS // _NBUF
_TAIL = _NCHUNKS - _ROUNDS * _NBUF


def _copy(hbm_ref, buf_ref, sem, chunk, slot):
    return pltpu.make_async_copy(
        hbm_ref.at[pl.ds(chunk * _CHUNK, _CHUNK), :],
        buf_ref.at[slot],
        sem.at[slot],
    )


def _arc_loss_kernel(last_ref, a_hbm, b_hbm, out_ref,
                     a_buf, b_buf, a_sem, b_sem):
    for s in range(_NBUF):
        _copy(a_hbm, a_buf, a_sem, s, s).start()
        _copy(b_hbm, b_buf, b_sem, s, s).start()

    def process(g, slot, acc):
        _copy(a_hbm, a_buf, a_sem, g, slot).wait()
        _copy(b_hbm, b_buf, b_sem, g, slot).wait()
        t = a_buf[slot] * b_buf[slot]
        part = jnp.sum((t * t).reshape(_CHUNK // 8, 8, _D), axis=0)

        nxt = g + _NBUF

        @pl.when(nxt < _NCHUNKS)
        def _refill():
            _copy(a_hbm, a_buf, a_sem, nxt, slot).start()
            _copy(b_hbm, b_buf, b_sem, nxt, slot).start()

        return acc + part

    def round_body(r, acc):
        for s in range(_NBUF):
            acc = process(r * _NBUF + s, s, acc)
        return acc

    acc = jax.lax.fori_loop(
        0, _ROUNDS, round_body, jnp.zeros((8, _D), jnp.float32))
    for s in range(_TAIL):
        acc = process(_ROUNDS * _NBUF + s, s, acc)

    denom = (last_ref[0] + 1).astype(jnp.float32)
    out_ref[...] = (jnp.sum(acc) / denom).reshape(1, 1)


def kernel(dx_dt, d2x_dt2, batch):
    last = batch[-1:].astype(jnp.int32)

    out = pl.pallas_call(
        _arc_loss_kernel,
        in_specs=[
            pl.BlockSpec(memory_space=pltpu.MemorySpace.SMEM),
            pl.BlockSpec(memory_space=pltpu.MemorySpace.HBM),
            pl.BlockSpec(memory_space=pltpu.MemorySpace.HBM),
        ],
        out_specs=pl.BlockSpec(memory_space=pltpu.MemorySpace.VMEM),
        out_shape=jax.ShapeDtypeStruct((1, 1), jnp.float32),
        scratch_shapes=[
            pltpu.VMEM((_NBUF, _CHUNK, _D), jnp.float32),
            pltpu.VMEM((_NBUF, _CHUNK, _D), jnp.float32),
            pltpu.SemaphoreType.DMA((_NBUF,)),
            pltpu.SemaphoreType.DMA((_NBUF,)),
        ],
    )(last, dx_dt, d2x_dt2)
    return out[0, 0]


# speedup vs baseline: 1.0842x; 1.0842x over previous
import jax
import jax.numpy as jnp
from jax.experimental import pallas as pl
from jax.experimental.pallas import tpu as pltpu

_N = 100000
_D = 128
_CHUNK = 2000
_NBUF = 4
_NCHUNKS = _N // _CHUNK
_ROUNDS = _NCHUNKS // _NBUF
_TAIL = _NCHUNKS - _ROUNDS * _NBUF


def _copy(hbm_ref, buf_ref, sem, chunk, slot):
    return pltpu.make_async_copy(
        hbm_ref.at[pl.ds(chunk * _CHUNK, _CHUNK), :],
        buf_ref.at[slot],
        sem.at[slot],
    )


def _arc_loss_kernel(last_ref, a_hbm, b_hbm, out_ref,
                     a_buf, b_buf, a_sem, b_sem):
    for s in range(_NBUF):
        _copy(a_hbm, a_buf, a_sem, s, s).start()
        _copy(b_hbm, b_buf, b_sem, s, s).start()

    def process(g, slot, acc):
        _copy(a_hbm, a_buf, a_sem, g, slot).wait()
        _copy(b_hbm, b_buf, b_sem, g, slot).wait()
        t = a_buf[slot] * b_buf[slot]
        part = jnp.sum((t * t).reshape(_CHUNK // 8, 8, _D), axis=0)

        nxt = g + _NBUF

        @pl.when(nxt < _NCHUNKS)
        def _refill():
            _copy(a_hbm, a_buf, a_sem, nxt, slot).start()
            _copy(b_hbm, b_buf, b_sem, nxt, slot).start()

        return acc + part

    def round_body(r, acc):
        for s in range(_NBUF):
            acc = process(r * _NBUF + s, s, acc)
        return acc

    acc = jax.lax.fori_loop(
        0, _ROUNDS, round_body, jnp.zeros((8, _D), jnp.float32))
    for s in range(_TAIL):
        acc = process(_ROUNDS * _NBUF + s, s, acc)

    denom = (last_ref[0] + 1).astype(jnp.float32)
    out_ref[...] = (jnp.sum(acc) / denom).reshape(1, 1)


def kernel(dx_dt, d2x_dt2, batch):
    last = batch[-1:].astype(jnp.int32)

    out = pl.pallas_call(
        _arc_loss_kernel,
        in_specs=[
            pl.BlockSpec(memory_space=pltpu.MemorySpace.SMEM),
            pl.BlockSpec(memory_space=pltpu.MemorySpace.HBM),
            pl.BlockSpec(memory_space=pltpu.MemorySpace.HBM),
        ],
        out_specs=pl.BlockSpec(memory_space=pltpu.MemorySpace.VMEM),
        out_shape=jax.ShapeDtypeStruct((1, 1), jnp.float32),
        scratch_shapes=[
            pltpu.VMEM((_NBUF, _CHUNK, _D), jnp.float32),
            pltpu.VMEM((_NBUF, _CHUNK, _D), jnp.float32),
            pltpu.SemaphoreType.DMA((_NBUF,)),
            pltpu.SemaphoreType.DMA((_NBUF,)),
        ],
    )(last, dx_dt, d2x_dt2)
    return out[0, 0]
